# baseline (speedup 1.0000x reference)
import jax
import jax.numpy as jnp
from jax import lax
from jax.experimental import pallas as pl
from jax.experimental.pallas import tpu as pltpu

N = 4096
E = 8192
B = 16
T = 512
DIN = 8
H = 128
DE = 4
V = 5


def _prep_body(x_ref, ea_ref, w_ref, we_ref, h_ref, ef_ref):
    h_ref[...] = jnp.dot(x_ref[...], w_ref[...],
                         preferred_element_type=jnp.float32)
    ef_ref[...] = jnp.dot(ea_ref[...], we_ref[...],
                          preferred_element_type=jnp.float32)


def _prep(x, edge_attr, W, We):
    return pl.pallas_call(
        _prep_body,
        out_shape=(
            jax.ShapeDtypeStruct((N, H), jnp.float32),
            jax.ShapeDtypeStruct((E, H), jnp.float32),
        ),
    )(x, edge_attr, W, We)


def _dec_body(enc_ref, tgt_ref, sizes_ref, emb_ref, wz_ref, uz_ref, bz_ref,
              wr_ref, ur_ref, br_ref, wh_ref, uh_ref, bh_ref, wo_ref, bo_ref,
              logits_ref, mask_ref):
    sizes = sizes_ref[...]
    t_iota = lax.broadcasted_iota(jnp.int32, (B, T), 1)
    mask = t_iota < sizes
    mask_ref[...] = mask.astype(jnp.int32)
    maskf = mask.astype(jnp.float32)
    neg = (1.0 - maskf) * (-1e9)

    enc2d = enc_ref[...]
    i_wide = lax.broadcasted_iota(jnp.int32, (B, B * T), 1)
    b_wide = lax.broadcasted_iota(jnp.int32, (B, B * T), 0)
    sel_wide = ((i_wide // T) == b_wide).astype(jnp.float32)
    msum = jnp.sum(maskf, axis=1, keepdims=True)
    enc3d = enc2d.reshape(B, T, H)
    h0 = (jnp.sum(enc3d * maskf[:, :, None], axis=1)
          / (msum + 1e-9))

    eye = (lax.broadcasted_iota(jnp.int32, (B, B), 0)
           == lax.broadcasted_iota(jnp.int32, (B, B), 1)).astype(jnp.float32)
    emb = emb_ref[...]
    wz, uz, bz = wz_ref[...], uz_ref[...], bz_ref[...]
    wr, ur, br = wr_ref[...], ur_ref[...], br_ref[...]
    wh, uh, bh = wh_ref[...], uh_ref[...], bh_ref[...]
    wo, bo = wo_ref[...], bo_ref[...]

    def step(t, hprev):
        tprev_idx = jnp.maximum(t - 1, 0)
        tp = tgt_ref[pl.ds(tprev_idx, 1)].reshape(1, B)
        tp = jnp.where(t == 0, 0, tp)
        tp_col = jax.lax.dot_general(
            eye, tp.astype(jnp.float32), (((1,), (1,)), ((), ())),
            preferred_element_type=jnp.float32)
        e_t = jnp.zeros((B, H), jnp.float32)
        for v in range(V):
            e_t = e_t + ((tp_col == float(v)).astype(jnp.float32)
                         * emb[v, :][None, :])

        s_full = jax.lax.dot_general(
            enc2d, hprev, (((1,), (1,)), ((), ())),
            preferred_element_type=jnp.float32)
        scores = jnp.sum(s_full.reshape(B, T, B) * eye[:, None, :],
                         axis=2) + neg
        m = jnp.max(scores, axis=1, keepdims=True)
        ex = jnp.exp(scores - m)
        att = ex / jnp.sum(ex, axis=1, keepdims=True)
        att_wide = sel_wide * jnp.tile(att, (1, B))
        ctx = jnp.dot(att_wide, enc2d,
                      preferred_element_type=jnp.float32)

        inp = jnp.concatenate([e_t, ctx], axis=1)
        z = jax.nn.sigmoid(
            jnp.dot(inp, wz, preferred_element_type=jnp.float32)
            + jnp.dot(hprev, uz, preferred_element_type=jnp.float32) + bz)
        r = jax.nn.sigmoid(
            jnp.dot(inp, wr, preferred_element_type=jnp.float32)
            + jnp.dot(hprev, ur, preferred_element_type=jnp.float32) + br)
        hh = jnp.tanh(
            jnp.dot(inp, wh, preferred_element_type=jnp.float32)
            + jnp.dot(r * hprev, uh, preferred_element_type=jnp.float32) + bh)
        hnew = (1.0 - z) * hprev + z * hh
        out_t = jnp.dot(jnp.concatenate([hnew, ctx], axis=1), wo,
                        preferred_element_type=jnp.float32) + bo
        logits_ref[pl.ds(t, 1), :, :] = out_t[None, :, :]
        return hnew

    lax.fori_loop(0, T, step, h0)


def _decode(enc2d, targets_t1b, sizes, emb, Wz, Uz, bz, Wr, Ur, br, Wh, Uh,
            bh, Wo, bo):
    return pl.pallas_call(
        _dec_body,
        out_shape=(
            jax.ShapeDtypeStruct((T, B, V), jnp.float32),
            jax.ShapeDtypeStruct((B, T), jnp.int32),
        ),
    )(enc2d, targets_t1b, sizes, emb, Wz, Uz, bz.reshape(1, H), Wr, Ur,
      br.reshape(1, H), Wh, Uh, bh.reshape(1, H), Wo, bo.reshape(1, V))


def kernel(x, edge_index, edge_attr, y, batch_vec, W, We, a_src, a_dst,
           a_edge, emb, Wz, Uz, bz, Wr, Ur, br, Wh, Uh, bh, Wo, bo):
    h = x @ W
    ef = edge_attr @ We

    src = edge_index[0]
    dst = edge_index[1]
    logit = jax.nn.leaky_relu(
        h[src] @ a_src + h[dst] @ a_dst + ef @ a_edge, 0.2)
    m = jax.ops.segment_max(logit, dst, num_segments=N)
    ex = jnp.exp(logit - m[dst])
    den = jax.ops.segment_sum(ex, dst, num_segments=N)
    alpha = ex / (den[dst] + 1e-9)
    node_emb = jax.ops.segment_sum(alpha[:, None] * h[src], dst,
                                   num_segments=N)
    node_emb = jax.nn.elu(node_emb)

    sizes = jnp.bincount(batch_vec, length=B)
    offsets = jnp.concatenate(
        [jnp.zeros((1,), sizes.dtype), jnp.cumsum(sizes)[:-1]])
    pos = jnp.arange(N) - offsets[batch_vec]
    enc = jnp.zeros((B, T, H), node_emb.dtype).at[batch_vec, pos].set(
        node_emb, mode='drop')
    targets = jnp.zeros((B, T), jnp.int32).at[batch_vec, pos].set(
        y.astype(jnp.int32), mode='drop')

    enc2d = enc.reshape(B * T, H)
    targets_t1b = jnp.transpose(targets, (1, 0)).reshape(T, 1, B)
    logits_tbv, mask_i32 = _decode(
        enc2d, targets_t1b, sizes.astype(jnp.int32).reshape(B, 1), emb, Wz,
        Uz, bz, Wr, Ur, br, Wh, Uh, bh, Wo, bo)
    logits = jnp.transpose(logits_tbv, (1, 0, 2))
    return logits, targets, mask_i32.astype(bool)


# speedup vs baseline: 4.1214x; 4.1214x over previous
import jax
import jax.numpy as jnp
from jax import lax
from jax.experimental import pallas as pl
from jax.experimental.pallas import tpu as pltpu

N = 4096
E = 8192
B = 16
T = 512
DIN = 8
H = 128
DE = 4
V = 5


def _dec_body(enc_ref, encT_ref, tgt_ref, sizes_ref, emb_ref, wz_ref, uz_ref,
              bz_ref, wr_ref, ur_ref, br_ref, wh_ref, uh_ref, bh_ref, wo_ref,
              bo_ref, logits_ref, mask_ref):
    sizes = sizes_ref[...]
    t_iota = lax.broadcasted_iota(jnp.int32, (B, T), 1)
    mask = t_iota < sizes
    mask_ref[...] = mask.astype(jnp.int32)
    maskf = mask.astype(jnp.float32)
    neg = (1.0 - maskf) * (-1e9)

    enc2d = enc_ref[...]
    encT = encT_ref[...]
    msum = jnp.sum(maskf, axis=1, keepdims=True)
    enc3d = enc2d.reshape(B, T, H)
    h0 = (jnp.sum(enc3d * maskf[:, :, None], axis=1)
          / (msum + 1e-9))

    eye = (lax.broadcasted_iota(jnp.int32, (B, B), 0)
           == lax.broadcasted_iota(jnp.int32, (B, B), 1)).astype(jnp.float32)
    emb = emb_ref[...]
    wz, uz, bz = wz_ref[...], uz_ref[...], bz_ref[...]
    wr, ur, br = wr_ref[...], ur_ref[...], br_ref[...]
    wh, uh, bh = wh_ref[...], uh_ref[...], bh_ref[...]
    wo, bo = wo_ref[...], bo_ref[...]

    def step(t, hprev):
        tprev_idx = jnp.maximum(t - 1, 0)
        tp = tgt_ref[pl.ds(tprev_idx, 1)].reshape(1, B)
        tp = jnp.where(t == 0, 0, tp)
        tp_col = jax.lax.dot_general(
            eye, tp.astype(jnp.float32), (((1,), (1,)), ((), ())),
            preferred_element_type=jnp.float32)
        e_t = jnp.zeros((B, H), jnp.float32)
        for v in range(V):
            e_t = e_t + ((tp_col == float(v)).astype(jnp.float32)
                         * emb[v, :][None, :])

        scores = jnp.concatenate(
            [jnp.dot(hprev[b:b + 1, :], encT[:, b * T:(b + 1) * T],
                     preferred_element_type=jnp.float32)
             for b in range(B)], axis=0) + neg
        m = jnp.max(scores, axis=1, keepdims=True)
        ex = jnp.exp(scores - m)
        att = ex / jnp.sum(ex, axis=1, keepdims=True)
        ctx = jnp.concatenate(
            [jnp.dot(att[b:b + 1, :], enc2d[b * T:(b + 1) * T, :],
                     preferred_element_type=jnp.float32)
             for b in range(B)], axis=0)

        inp = jnp.concatenate([e_t, ctx], axis=1)
        z = jax.nn.sigmoid(
            jnp.dot(inp, wz, preferred_element_type=jnp.float32)
            + jnp.dot(hprev, uz, preferred_element_type=jnp.float32) + bz)
        r = jax.nn.sigmoid(
            jnp.dot(inp, wr, preferred_element_type=jnp.float32)
            + jnp.dot(hprev, ur, preferred_element_type=jnp.float32) + br)
        hh = jnp.tanh(
            jnp.dot(inp, wh, preferred_element_type=jnp.float32)
            + jnp.dot(r * hprev, uh, preferred_element_type=jnp.float32) + bh)
        hnew = (1.0 - z) * hprev + z * hh
        out_t = jnp.dot(jnp.concatenate([hnew, ctx], axis=1), wo,
                        preferred_element_type=jnp.float32) + bo
        logits_ref[pl.ds(t, 1), :, :] = out_t[None, :, :]
        return hnew

    lax.fori_loop(0, T, step, h0)


def _decode(enc2d, encT, targets_t1b, sizes, emb, Wz, Uz, bz, Wr, Ur, br,
            Wh, Uh, bh, Wo, bo):
    return pl.pallas_call(
        _dec_body,
        out_shape=(
            jax.ShapeDtypeStruct((T, B, V), jnp.float32),
            jax.ShapeDtypeStruct((B, T), jnp.int32),
        ),
    )(enc2d, encT, targets_t1b, sizes, emb, Wz, Uz, bz.reshape(1, H), Wr,
      Ur, br.reshape(1, H), Wh, Uh, bh.reshape(1, H), Wo, bo.reshape(1, V))


def kernel(x, edge_index, edge_attr, y, batch_vec, W, We, a_src, a_dst,
           a_edge, emb, Wz, Uz, bz, Wr, Ur, br, Wh, Uh, bh, Wo, bo):
    h = x @ W
    ef = edge_attr @ We

    src = edge_index[0]
    dst = edge_index[1]
    logit = jax.nn.leaky_relu(
        h[src] @ a_src + h[dst] @ a_dst + ef @ a_edge, 0.2)
    m = jax.ops.segment_max(logit, dst, num_segments=N)
    ex = jnp.exp(logit - m[dst])
    den = jax.ops.segment_sum(ex, dst, num_segments=N)
    alpha = ex / (den[dst] + 1e-9)
    node_emb = jax.ops.segment_sum(alpha[:, None] * h[src], dst,
                                   num_segments=N)
    node_emb = jax.nn.elu(node_emb)

    sizes = jnp.bincount(batch_vec, length=B)
    offsets = jnp.concatenate(
        [jnp.zeros((1,), sizes.dtype), jnp.cumsum(sizes)[:-1]])
    pos = jnp.arange(N) - offsets[batch_vec]
    enc = jnp.zeros((B, T, H), node_emb.dtype).at[batch_vec, pos].set(
        node_emb, mode='drop')
    targets = jnp.zeros((B, T), jnp.int32).at[batch_vec, pos].set(
        y.astype(jnp.int32), mode='drop')

    enc2d = enc.reshape(B * T, H)
    encT = jnp.transpose(enc2d)
    targets_t1b = jnp.transpose(targets, (1, 0)).reshape(T, 1, B)
    logits_tbv, mask_i32 = _decode(
        enc2d, encT, targets_t1b, sizes.astype(jnp.int32).reshape(B, 1),
        emb, Wz, Uz, bz, Wr, Ur, br, Wh, Uh, bh, Wo, bo)
    logits = jnp.transpose(logits_tbv, (1, 0, 2))
    return logits, targets, mask_i32.astype(bool)
